# baseline (speedup 1.0000x reference)
import functools
import math

import jax
import jax.numpy as jnp
from jax import lax
from jax.experimental import pallas as pl
from jax.experimental.pallas import tpu as pltpu
from jax.experimental.pallas import tpu_sc as plsc

VOCAB = 1000000
D = 64
SCALE = math.sqrt(D)
NC, NS = 2, 16
NW = NC * NS
CH = 200

TOT_A = VOCAB // CH
NCH_A = -(-TOT_A // NW)


def _pipe2(nch, start_in, wait_in, proc, start_out, wait_out):
    def body(c, b, head, tail):
        wait_in(c, b)
        if not head:
            wait_out(c - 2, b)
        proc(c, b)
        start_out(c, b)
        if not tail:
            start_in(c + 2, b)

    start_in(0, 0)
    start_in(1, 1)
    body(0, 0, True, False)
    body(1, 1, True, False)
    main = nch - 4
    m, rem = divmod(main, 2)

    if m > 0:
        def outer(t, _):
            c0 = 2 + t * 2
            body(c0, 0, False, False)
            body(c0 + 1, 1, False, False)
            return 0
        lax.fori_loop(0, m, outer, 0)
    for i in range(rem):
        c = 2 + 2 * m + i
        body(c, c % 2, False, False)
    body(nch - 2, (nch - 2) % 2, False, True)
    body(nch - 1, (nch - 1) % 2, False, True)
    wait_out(nch - 2, (nch - 2) % 2)
    wait_out(nch - 1, (nch - 1) % 2)


def _emb_body(B, BPW,
              x_hbm, table_hbm, out_hbm, scratch_hbm,
              idx_all, u0, u1, w0, w1,
              g0, g1, o0, o1, bsem):
    nar = (u0, u1)
    wide = (w0, w1)
    gsem = (g0, g1)
    osem = (o0, o1)
    cidx = lax.axis_index("c")
    sid = lax.axis_index("s")
    wid = sid * NC + cidx
    base = wid * BPW
    nch_b = BPW // CH
    x1base = wid * nch_b

    pltpu.sync_copy(x_hbm.at[pl.ds(base, BPW)], idx_all)

    def a_off(k):
        return lax.rem(wid + k * NW, TOT_A) * CH

    def a_start_in(k, b):
        pltpu.async_copy(table_hbm.at[pl.ds(a_off(k), CH), :], nar[b],
                         gsem[b])

    def a_wait_in(k, b):
        pltpu.make_async_copy(table_hbm.at[pl.ds(a_off(k), CH), :], nar[b],
                              gsem[b]).wait()

    def a_proc(k, b):
        src, dst = nar[b], wide[b]

        @plsc.parallel_loop(0, CH, unroll=4)
        def _(i):
            for col in range(D // 16):
                sl = pl.ds(col * 16, 16)
                dst[i, sl] = src[i, sl]

    def a_start_out(k, b):
        pltpu.async_copy(wide[b], scratch_hbm.at[pl.ds(a_off(k), CH), :],
                         osem[b])

    def a_wait_out(k, b):
        pltpu.make_async_copy(wide[b], scratch_hbm.at[pl.ds(a_off(k), CH), :],
                              osem[b]).wait()

    _pipe2(NCH_A, a_start_in, a_wait_in, a_proc, a_start_out, a_wait_out)

    plsc.subcore_barrier()
    pl.semaphore_signal(bsem, 1, core_index=1 - cidx)
    pl.semaphore_wait(bsem, 1)

    def b_start_in(c, b):
        pltpu.async_copy(scratch_hbm.at[idx_all.at[pl.ds(c * CH, CH)]],
                         wide[b], gsem[b])

    def b_wait_in(c, b):
        pltpu.make_async_copy(scratch_hbm.at[idx_all.at[pl.ds(c * CH, CH)]],
                              wide[b], gsem[b]).wait()

    def b_proc(c, b):
        src, dst = wide[b], nar[b]

        @plsc.parallel_loop(0, CH, unroll=4)
        def _(i):
            for col in range(D // 16):
                sl = pl.ds(col * 16, 16)
                dst[i, sl] = src[i, sl] * SCALE

    def b_start_out(c, b):
        pltpu.async_copy(nar[b], out_hbm.at[x1base + c], osem[b])

    def b_wait_out(c, b):
        pltpu.make_async_copy(nar[b], out_hbm.at[x1base + c], osem[b]).wait()

    _pipe2(nch_b, b_start_in, b_wait_in, b_proc, b_start_out, b_wait_out)


@functools.partial(jax.jit, static_argnames=("B", "R"))
def _emb(xf, table, B, R):
    BPW = B // NW
    body = functools.partial(_emb_body, B, BPW)
    run = pl.kernel(
        body,
        out_type=(jax.ShapeDtypeStruct((R, CH, D), jnp.float32),
                  jax.ShapeDtypeStruct((VOCAB, 2 * D), jnp.float32)),
        mesh=plsc.VectorSubcoreMesh(core_axis_name="c", subcore_axis_name="s",
                                    num_cores=NC, num_subcores=NS),
        scratch_types=[
            pltpu.VMEM((BPW,), jnp.int32),
            pltpu.VMEM((CH, D), jnp.float32),
            pltpu.VMEM((CH, D), jnp.float32),
            pltpu.VMEM((CH, 2 * D), jnp.float32),
            pltpu.VMEM((CH, 2 * D), jnp.float32),
            pltpu.SemaphoreType.DMA,
            pltpu.SemaphoreType.DMA,
            pltpu.SemaphoreType.DMA,
            pltpu.SemaphoreType.DMA,
            pltpu.SemaphoreType.REGULAR,
        ],
    )
    out, _ = run(xf, table)
    return out


def kernel(x, table):
    R, C = x.shape
    assert C == CH and table.shape == (VOCAB, D)
    B = R * C
    xf = x.reshape(B).astype(jnp.int32)
    out = _emb(xf, table, B, R)
    return out.reshape(R, C, D)


# speedup vs baseline: 1.0515x; 1.0515x over previous
import functools
import math

import jax
import jax.numpy as jnp
from jax import lax
from jax.experimental import pallas as pl
from jax.experimental.pallas import tpu as pltpu
from jax.experimental.pallas import tpu_sc as plsc

VOCAB = 1000000
D = 64
SCALE = math.sqrt(D)
NC, NS = 2, 16
NW = NC * NS
CH = 200
NBUF = 3


def _emb_body(B, BPW, NCHUNK,
              x_hbm, table_hbm, out_hbm,
              idx_all, r0, r1, r2, g0, g1, g2, o0, o1, o2):
    rows = (r0, r1, r2)
    gsem = (g0, g1, g2)
    osem = (o0, o1, o2)
    wid = lax.axis_index("s") * NC + lax.axis_index("c")
    base = wid * BPW
    x1base = wid * NCHUNK

    pltpu.sync_copy(x_hbm.at[pl.ds(base, BPW)], idx_all)

    def start_gather(c, b):
        pltpu.async_copy(table_hbm.at[idx_all.at[pl.ds(c * CH, CH)]],
                         rows[b], gsem[b])

    def wait_gather(c, b):
        pltpu.make_async_copy(table_hbm.at[idx_all.at[pl.ds(c * CH, CH)]],
                              rows[b], gsem[b]).wait()

    def scale(b):
        r = rows[b]

        @plsc.parallel_loop(0, CH, unroll=4)
        def _(i):
            for col in range(D // 16):
                sl = (i, pl.ds(col * 16, 16))
                r[sl] = r[sl] * SCALE

    def start_out(c, b):
        pltpu.async_copy(rows[b], out_hbm.at[x1base + c], osem[b])

    def wait_out(c, b):
        pltpu.make_async_copy(rows[b], out_hbm.at[x1base + c],
                              osem[b]).wait()

    def drain_body(c, b):
        wait_gather(c, b)
        scale(b)
        start_out(c, b)

    def uniform_body(c, b):
        bb = (b + 2) % NBUF
        wait_out(c - 1, bb)
        start_gather(c + 2, bb)
        drain_body(c, b)

    start_gather(0, 0)
    start_gather(1, 1)
    drain_body(0, 0)
    start_gather(2, 2)
    uniform_body(1, 1)

    n_uni = NCHUNK - 5
    n3, rem = divmod(n_uni, NBUF)

    if n3 > 0:
        def outer(t, _):
            c0 = 2 + t * NBUF
            for j in range(NBUF):
                uniform_body(c0 + j, (2 + j) % NBUF)
            return 0
        lax.fori_loop(0, n3, outer, 0)
    for i in range(rem):
        c = 2 + n3 * NBUF + i
        uniform_body(c, c % NBUF)

    uniform_body(NCHUNK - 3, (NCHUNK - 3) % NBUF)
    drain_body(NCHUNK - 2, (NCHUNK - 2) % NBUF)
    drain_body(NCHUNK - 1, (NCHUNK - 1) % NBUF)
    for c in (NCHUNK - 3, NCHUNK - 2, NCHUNK - 1):
        wait_out(c, c % NBUF)


@functools.partial(jax.jit, static_argnames=("B", "R"))
def _emb(xf, table, B, R):
    BPW = B // NW
    NCHUNK = BPW // CH
    body = functools.partial(_emb_body, B, BPW, NCHUNK)
    run = pl.kernel(
        body,
        out_type=jax.ShapeDtypeStruct((R, CH, D), jnp.float32),
        mesh=plsc.VectorSubcoreMesh(core_axis_name="c", subcore_axis_name="s",
                                    num_cores=NC, num_subcores=NS),
        compiler_params=pltpu.CompilerParams(use_tc_tiling_on_sc=False),
        scratch_types=[
            pltpu.VMEM((BPW,), jnp.int32),
            pltpu.VMEM((CH, D), jnp.float32),
            pltpu.VMEM((CH, D), jnp.float32),
            pltpu.VMEM((CH, D), jnp.float32),
            pltpu.SemaphoreType.DMA,
            pltpu.SemaphoreType.DMA,
            pltpu.SemaphoreType.DMA,
            pltpu.SemaphoreType.DMA,
            pltpu.SemaphoreType.DMA,
            pltpu.SemaphoreType.DMA,
        ],
    )
    return run(xf, table)


def kernel(x, table):
    R, C = x.shape
    assert C == CH and table.shape == (VOCAB, D)
    B = R * C
    xf = x.reshape(B).astype(jnp.int32)
    out = _emb(xf, table, B, R)
    return out.reshape(R, C, D)


# speedup vs baseline: 1.2784x; 1.2158x over previous
import functools
import math

import jax
import jax.numpy as jnp
from jax import lax
from jax.experimental import pallas as pl
from jax.experimental.pallas import tpu as pltpu
from jax.experimental.pallas import tpu_sc as plsc

VOCAB = 1000000
D = 64
SCALE = math.sqrt(D)
NC, NS = 2, 16
NW = NC * NS
CH = 200


def _emb_body(B, BPW, NCHUNK,
              x_hbm, table_hbm, out_hbm,
              idx_all, n0, n1, n2, w0, w1,
              g0, g1, g2, o0, o1):
    nar = (n0, n1, n2)
    wide = (w0, w1)
    gsem = (g0, g1, g2)
    osem = (o0, o1)
    wid = lax.axis_index("s") * NC + lax.axis_index("c")
    base = wid * BPW
    x1base = wid * NCHUNK

    pltpu.sync_copy(x_hbm.at[pl.ds(base, BPW)], idx_all)

    def start_gather(c, b):
        pltpu.async_copy(table_hbm.at[idx_all.at[pl.ds(c * CH, CH)]],
                         nar[b], gsem[b])

    def wait_gather(c, b):
        pltpu.make_async_copy(table_hbm.at[idx_all.at[pl.ds(c * CH, CH)]],
                              nar[b], gsem[b]).wait()

    def scale(b, v):
        src, dst = nar[b], wide[v]

        @plsc.parallel_loop(0, CH, unroll=4)
        def _(i):
            for col in range(D // 16):
                sl = pl.ds(col * 16, 16)
                dst[i, sl] = src[i, sl] * SCALE

    def start_out(c, v):
        pltpu.async_copy(wide[v], out_hbm.at[x1base + c], osem[v])

    def wait_out(c, v):
        pltpu.make_async_copy(wide[v], out_hbm.at[x1base + c],
                              osem[v]).wait()

    def body(c, head, tail, b, v):
        wait_gather(c, b)
        if not head:
            wait_out(c - 2, v)
        scale(b, v)
        start_out(c, v)
        if not tail:
            start_gather(c + 3, b)

    start_gather(0, 0)
    start_gather(1, 1)
    start_gather(2, 2)
    body(0, True, False, 0, 0)
    body(1, True, False, 1, 1)
    lo = 2
    hi = NCHUNK - 3
    n6, rem = divmod(hi - lo, 6)

    if n6 > 0:
        def outer(t, _):
            c0 = lo + t * 6
            for j in range(6):
                body(c0 + j, False, False, (lo + j) % 3, (lo + j) % 2)
            return 0
        lax.fori_loop(0, n6, outer, 0)
    for c in range(lo + n6 * 6, hi):
        body(c, False, False, c % 3, c % 2)
    for c in range(hi, NCHUNK):
        body(c, False, True, c % 3, c % 2)
    wait_out(NCHUNK - 2, (NCHUNK - 2) % 2)
    wait_out(NCHUNK - 1, (NCHUNK - 1) % 2)


@functools.partial(jax.jit, static_argnames=("B", "R"))
def _emb(xf, table, B, R):
    BPW = B // NW
    NCHUNK = BPW // CH
    body = functools.partial(_emb_body, B, BPW, NCHUNK)
    run = pl.kernel(
        body,
        out_type=jax.ShapeDtypeStruct((R, CH, 2 * D), jnp.float32),
        mesh=plsc.VectorSubcoreMesh(core_axis_name="c", subcore_axis_name="s",
                                    num_cores=NC, num_subcores=NS),
        compiler_params=pltpu.CompilerParams(use_tc_tiling_on_sc=False),
        scratch_types=[
            pltpu.VMEM((BPW,), jnp.int32),
            pltpu.VMEM((CH, D), jnp.float32),
            pltpu.VMEM((CH, D), jnp.float32),
            pltpu.VMEM((CH, D), jnp.float32),
            pltpu.VMEM((CH, 2 * D), jnp.float32),
            pltpu.VMEM((CH, 2 * D), jnp.float32),
            pltpu.SemaphoreType.DMA,
            pltpu.SemaphoreType.DMA,
            pltpu.SemaphoreType.DMA,
            pltpu.SemaphoreType.DMA,
            pltpu.SemaphoreType.DMA,
        ],
    )
    return run(xf, table)


def kernel(x, table):
    R, C = x.shape
    assert C == CH and table.shape == (VOCAB, D)
    B = R * C
    xf = x.reshape(B).astype(jnp.int32)
    out = _emb(xf, table, B, R)
    return out[:, :, :D]
